# Initial kernel scaffold; baseline (speedup 1.0000x reference)
import jax
import jax.numpy as jnp
from jax.experimental import pallas as pl


def kernel(story, query, A0, C0, C1, C2, TA, TC):
    raise NotImplementedError("write your pallas kernel here")


# speedup vs baseline: 8.7742x; 8.7742x over previous
import functools

import jax
import jax.numpy as jnp
from jax import lax
from jax.experimental import pallas as pl
from jax.experimental.pallas import tpu as pltpu
from jax.experimental.pallas import tpu_sc as plsc

VOCAB = 100000
EMBED = 64
MEM = 50
SEN = 20
BATCH = 1024
NSEG = BATCH * MEM

NC, NS = 2, 16
NW = NC * NS
SEG_PER_W = NSEG // NW
CHUNK = 64
NCHUNK = SEG_PER_W // CHUNK
ROWS_PER_CHUNK = CHUNK * SEN
IDXROWS_PER_CHUNK = ROWS_PER_CHUNK // 128
Q_PER_W = BATCH // NW


def _pos_weights(J, d):
    j = jnp.arange(J, dtype=jnp.float32)[:, None]
    k = jnp.arange(d, dtype=jnp.float32)[None, :]
    return 1.0 - (j + 1.0) / J - (k + 1.0) / d * (1.0 - 2.0 * (j + 1.0) / J)


def _sc_pool_body(idx_story, idx_query, ws_hbm, A0, C0, C1, C2,
                  outA, outC0, outC1, outC2, outQ,
                  idx_v, rows_v, pool_v, ws_v, sem):
    wid = lax.axis_index("s") * NC + lax.axis_index("c")
    pltpu.sync_copy(ws_hbm, ws_v)

    def do_chunk(tbl, idx_hbm, idx_off, out_hbm, seg_off, n_seg):
        n_idx = n_seg * SEN
        pltpu.sync_copy(idx_hbm.at[pl.ds(idx_off, n_idx)],
                        idx_v.at[pl.ds(0, n_idx)])
        cps = [
            pltpu.async_copy(tbl.at[idx_v.at[pl.ds(i * 128, 128)]],
                             rows_v.at[pl.ds(i * 128, 128)], sem)
            for i in range(n_idx // 128)
        ]
        for cp in cps:
            cp.wait()
        for k4 in range(EMBED // 16):
            sl = pl.ds(k4 * 16, 16)
            wsr = [ws_v[s, sl] for s in range(SEN)]

            def seg_body(c, _):
                acc = rows_v[c * SEN, sl] * wsr[0]
                for s in range(1, SEN):
                    acc = acc + rows_v[c * SEN + s, sl] * wsr[s]
                pool_v[c, sl] = acc
                return 0

            lax.fori_loop(0, n_seg, seg_body, 0)
        pltpu.sync_copy(pool_v.at[pl.ds(0, n_seg)],
                        out_hbm.at[pl.ds(seg_off, n_seg)])

    for tbl, out_hbm in ((A0, outA), (C0, outC0), (C1, outC1), (C2, outC2)):
        def chunk_body(j, _, tbl=tbl, out_hbm=out_hbm):
            do_chunk(tbl, idx_story,
                     wid * (SEG_PER_W * SEN) + j * ROWS_PER_CHUNK,
                     out_hbm, wid * SEG_PER_W + j * CHUNK, CHUNK)
            return 0

        lax.fori_loop(0, NCHUNK, chunk_body, 0)

    do_chunk(A0, idx_query, wid * (Q_PER_W * SEN),
             outQ, wid * Q_PER_W, Q_PER_W)


@functools.cache
def _sc_pool():
    return pl.kernel(
        _sc_pool_body,
        mesh=plsc.VectorSubcoreMesh(core_axis_name="c", subcore_axis_name="s"),
        compiler_params=pltpu.CompilerParams(use_tc_tiling_on_sc=False),
        out_type=[jax.ShapeDtypeStruct((NSEG, EMBED), jnp.float32)] * 4
        + [jax.ShapeDtypeStruct((BATCH, EMBED), jnp.float32)],
        scratch_types=[
            pltpu.VMEM((ROWS_PER_CHUNK,), jnp.int32),
            pltpu.VMEM((ROWS_PER_CHUNK, EMBED), jnp.float32),
            pltpu.VMEM((CHUNK, EMBED), jnp.float32),
            pltpu.VMEM((SEN, EMBED), jnp.float32),
            pltpu.SemaphoreType.DMA,
        ],
    )


RB = 128


def _hops_body(pa_ref, pc0_ref, pc1_ref, pc2_ref, st0_ref, ta_ref, tc_ref,
               out_ref):
    state = st0_ref[...]
    pools = (pa_ref[...], pc0_ref[...], pc1_ref[...], pc2_ref[...])
    ta = ta_ref[...]
    tc = tc_ref[...]
    for i in range(3):
        mem = pools[i] + ta
        outp = pools[i + 1] + tc
        logits = jnp.sum(mem * state[:, None, :], axis=-1)
        m = jnp.max(logits, axis=-1, keepdims=True)
        e = jnp.exp(logits - m)
        p = e / jnp.sum(e, axis=-1, keepdims=True)
        state = state + jnp.sum(p[:, :, None] * outp, axis=1)
    out_ref[...] = state


def _hops(pa, pc0, pc1, pc2, st0, TA, TC):
    pool_spec = pl.BlockSpec((RB, MEM, EMBED), lambda i: (i, 0, 0))
    return pl.pallas_call(
        _hops_body,
        grid=(BATCH // RB,),
        in_specs=[pool_spec, pool_spec, pool_spec, pool_spec,
                  pl.BlockSpec((RB, EMBED), lambda i: (i, 0)),
                  pl.BlockSpec((MEM, EMBED), lambda i: (0, 0)),
                  pl.BlockSpec((MEM, EMBED), lambda i: (0, 0))],
        out_specs=pl.BlockSpec((RB, EMBED), lambda i: (i, 0)),
        out_shape=jax.ShapeDtypeStruct((BATCH, EMBED), jnp.float32),
    )(pa, pc0, pc1, pc2, st0, TA, TC)


VB = 2048
NVB = -(-VOCAB // VB)


def _lse_body(state_ref, c2_ref, lse_ref, m_scr, s_scr):
    j = pl.program_id(0)
    x = lax.dot_general(state_ref[...], c2_ref[...],
                        (((1,), (1,)), ((), ())),
                        preferred_element_type=jnp.float32)
    col = j * VB + lax.broadcasted_iota(jnp.int32, x.shape, 1)
    x = jnp.where(col < VOCAB, x, -jnp.inf)

    @pl.when(j == 0)
    def _():
        m_scr[...] = jnp.full_like(m_scr, -jnp.inf)
        s_scr[...] = jnp.zeros_like(s_scr)

    m_old = m_scr[...]
    m_new = jnp.maximum(m_old, jnp.max(x, axis=-1, keepdims=True))
    s_new = s_scr[...] * jnp.exp(m_old - m_new) + \
        jnp.sum(jnp.exp(x - m_new), axis=-1, keepdims=True)
    m_scr[...] = m_new
    s_scr[...] = s_new
    lse_ref[...] = m_new + jnp.log(s_new)


def _lse(state, C2):
    return pl.pallas_call(
        _lse_body,
        grid=(NVB,),
        in_specs=[pl.BlockSpec((BATCH, EMBED), lambda j: (0, 0)),
                  pl.BlockSpec((VB, EMBED), lambda j: (j, 0))],
        out_specs=pl.BlockSpec((BATCH, 1), lambda j: (0, 0)),
        out_shape=jax.ShapeDtypeStruct((BATCH, 1), jnp.float32),
        scratch_shapes=[pltpu.VMEM((BATCH, 1), jnp.float32),
                        pltpu.VMEM((BATCH, 1), jnp.float32)],
    )(state, C2)


def _norm_body(state_ref, c2_ref, lse_ref, out_ref):
    x = lax.dot_general(state_ref[...], c2_ref[...],
                        (((1,), (1,)), ((), ())),
                        preferred_element_type=jnp.float32)
    out_ref[...] = x - lse_ref[...]


def _norm(state, C2, lse):
    return pl.pallas_call(
        _norm_body,
        grid=(NVB,),
        in_specs=[pl.BlockSpec((BATCH, EMBED), lambda j: (0, 0)),
                  pl.BlockSpec((VB, EMBED), lambda j: (j, 0)),
                  pl.BlockSpec((BATCH, 1), lambda j: (0, 0))],
        out_specs=pl.BlockSpec((BATCH, VB), lambda j: (0, j)),
        out_shape=jax.ShapeDtypeStruct((BATCH, VOCAB), jnp.float32),
    )(state, C2, lse)


def kernel(story, query, A0, C0, C1, C2, TA, TC):
    idx_story = story.astype(jnp.int32).reshape(NSEG * SEN)
    idx_query = query.astype(jnp.int32).reshape(BATCH * SEN)
    ws = _pos_weights(SEN, EMBED)
    pa, pc0, pc1, pc2, st0 = _sc_pool()(idx_story, idx_query, ws,
                                        A0, C0, C1, C2)
    shape3 = (BATCH, MEM, EMBED)
    state = _hops(pa.reshape(shape3), pc0.reshape(shape3),
                  pc1.reshape(shape3), pc2.reshape(shape3), st0, TA, TC)
    lse = _lse(state, C2)
    return _norm(state, C2, lse)


# speedup vs baseline: 13.4185x; 1.5293x over previous
import functools

import jax
import jax.numpy as jnp
from jax import lax
from jax.experimental import pallas as pl
from jax.experimental.pallas import tpu as pltpu
from jax.experimental.pallas import tpu_sc as plsc

VOCAB = 100000
EMBED = 64
MEM = 50
SEN = 20
BATCH = 1024
NSEG = BATCH * MEM

NC, NS = 2, 16
NW = NC * NS
SEG_PER_W = NSEG // NW
BANK_SEGS = 16
BANK_ROWS = BANK_SEGS * SEN
GROWS = 80
GPB = BANK_ROWS // GROWS
NBATCH = SEG_PER_W // BANK_SEGS
Q_PER_W = BATCH // NW


def _pos_weights(J, d):
    j = jnp.arange(J, dtype=jnp.float32)[:, None]
    k = jnp.arange(d, dtype=jnp.float32)[None, :]
    return 1.0 - (j + 1.0) / J - (k + 1.0) / d * (1.0 - 2.0 * (j + 1.0) / J)


def _sc_pool_body(idx_story, idx_query, ws2, AB0, AB1,
                  out0, out1, outQ,
                  idx_v, rows0, rows1, pool_v, ws_v, sem0, sem1):
    wid = lax.axis_index("s") * NC + lax.axis_index("c")
    pltpu.sync_copy(ws2, ws_v)
    pltpu.sync_copy(idx_story.at[pl.ds(wid * (SEG_PER_W * SEN),
                                       SEG_PER_W * SEN)], idx_v)
    rows = (rows0, rows1)
    sems = (sem0, sem1)

    def fire(tbl, bank, t):
        for i in range(GPB):
            pltpu.async_copy(
                tbl.at[idx_v.at[pl.ds(t * BANK_ROWS + i * GROWS, GROWS)]],
                rows[bank].at[pl.ds(i * GROWS, GROWS)], sems[bank])

    def drain(tbl, bank):
        pltpu.make_async_copy(tbl.at[pl.ds(0, BANK_ROWS)], rows[bank],
                              sems[bank]).wait()

    def compute_write(bank, out_hbm, seg_off, n_seg):
        rv = rows[bank]
        for k8 in range(8):
            sl = pl.ds(k8 * 16, 16)
            wsr = [ws_v[s, sl] for s in range(SEN)]

            def seg_body(c, _):
                acc = rv[c * SEN, sl] * wsr[0]
                for s in range(1, SEN):
                    acc = acc + rv[c * SEN + s, sl] * wsr[s]
                pool_v[c, sl] = acc
                return 0

            lax.fori_loop(0, n_seg, seg_body, 0)
        pltpu.sync_copy(pool_v.at[pl.ds(0, n_seg)],
                        out_hbm.at[pl.ds(seg_off, n_seg)])

    for tbl, out_hbm in ((AB0, out0), (AB1, out1)):
        fire(tbl, 0, 0)

        def body2(j2, _, tbl=tbl, out_hbm=out_hbm):
            for b2 in range(2):
                t = j2 * 2 + b2

                @pl.when(t + 1 < NBATCH)
                def _(tbl=tbl, b2=b2, t=t):
                    fire(tbl, (b2 + 1) % 2, t + 1)

                drain(tbl, b2)
                compute_write(b2, out_hbm,
                              wid * SEG_PER_W + t * BANK_SEGS, BANK_SEGS)
            return 0

        lax.fori_loop(0, NBATCH // 2, body2, 0)

    pltpu.sync_copy(idx_query.at[pl.ds(wid * (Q_PER_W * SEN),
                                       Q_PER_W * SEN)],
                    idx_v.at[pl.ds(0, Q_PER_W * SEN)])
    for b2 in range(2):
        fire(AB0, b2, b2)
    for b2 in range(2):
        drain(AB0, b2)
        compute_write(b2, outQ, wid * Q_PER_W + b2 * BANK_SEGS, BANK_SEGS)


@functools.cache
def _sc_pool():
    return pl.kernel(
        _sc_pool_body,
        mesh=plsc.VectorSubcoreMesh(core_axis_name="c", subcore_axis_name="s"),
        out_type=[jax.ShapeDtypeStruct((NSEG, 128), jnp.float32)] * 2
        + [jax.ShapeDtypeStruct((BATCH, 128), jnp.float32)],
        scratch_types=[
            pltpu.VMEM((SEG_PER_W * SEN,), jnp.int32),
            pltpu.VMEM((BANK_ROWS, 128), jnp.float32),
            pltpu.VMEM((BANK_ROWS, 128), jnp.float32),
            pltpu.VMEM((BANK_SEGS, 128), jnp.float32),
            pltpu.VMEM((SEN, 128), jnp.float32),
            pltpu.SemaphoreType.DMA,
            pltpu.SemaphoreType.DMA,
        ],
    )


RB = 128


def _hops_body(p0_ref, p1_ref, pq_ref, ta_ref, tc_ref, out_ref):
    P0 = p0_ref[...].reshape(RB, MEM, 128)
    P1 = p1_ref[...].reshape(RB, MEM, 128)
    state = pq_ref[...][:, :EMBED]
    ta = ta_ref[...]
    tc = tc_ref[...]
    halves = (P0[..., :EMBED], P0[..., EMBED:], P1[..., :EMBED],
              P1[..., EMBED:])
    for i in range(3):
        mem = halves[i] + ta
        outp = halves[i + 1] + tc
        logits = jnp.sum(mem * state[:, None, :], axis=-1)
        m = jnp.max(logits, axis=-1, keepdims=True)
        e = jnp.exp(logits - m)
        p = e / jnp.sum(e, axis=-1, keepdims=True)
        state = state + jnp.sum(p[:, :, None] * outp, axis=1)
    out_ref[...] = state


def _hops(p0, p1, pq, TA, TC):
    pool_spec = pl.BlockSpec((RB * MEM, 128), lambda i: (i, 0))
    return pl.pallas_call(
        _hops_body,
        grid=(BATCH // RB,),
        in_specs=[pool_spec, pool_spec,
                  pl.BlockSpec((RB, 128), lambda i: (i, 0)),
                  pl.BlockSpec((MEM, EMBED), lambda i: (0, 0)),
                  pl.BlockSpec((MEM, EMBED), lambda i: (0, 0))],
        out_specs=pl.BlockSpec((RB, EMBED), lambda i: (i, 0)),
        out_shape=jax.ShapeDtypeStruct((BATCH, EMBED), jnp.float32),
    )(p0, p1, pq, TA, TC)


VB = 2048
NVB = -(-VOCAB // VB)


def _logits_body(state_ref, c2_ref, out_ref, m_scr, s_scr):
    ph = pl.program_id(0)
    j = pl.program_id(1)
    x = lax.dot_general(c2_ref[...], state_ref[...],
                        (((1,), (1,)), ((), ())),
                        preferred_element_type=jnp.float32)

    @pl.when(ph == 0)
    def _():
        row = j * VB + lax.broadcasted_iota(jnp.int32, x.shape, 0)
        xm = jnp.where(row < VOCAB, x, -jnp.inf)

        @pl.when(j == 0)
        def _():
            m_scr[...] = jnp.full_like(m_scr, -jnp.inf)
            s_scr[...] = jnp.zeros_like(s_scr)

        m_old = m_scr[...]
        m_new = jnp.maximum(m_old, jnp.max(xm, axis=0, keepdims=True))
        s_scr[...] = s_scr[...] * jnp.exp(m_old - m_new) + \
            jnp.sum(jnp.exp(xm - m_new), axis=0, keepdims=True)
        m_scr[...] = m_new

    @pl.when(ph == 1)
    def _():
        out_ref[...] = x - (m_scr[...] + jnp.log(s_scr[...]))


def _logits(state, C2):
    return pl.pallas_call(
        _logits_body,
        grid=(2, NVB),
        in_specs=[pl.BlockSpec((BATCH, EMBED), lambda ph, j: (0, 0)),
                  pl.BlockSpec((VB, EMBED), lambda ph, j: (j, 0))],
        out_specs=pl.BlockSpec((VB, BATCH), lambda ph, j: (ph * j, 0)),
        out_shape=jax.ShapeDtypeStruct((VOCAB, BATCH), jnp.float32),
        scratch_shapes=[pltpu.VMEM((1, BATCH), jnp.float32),
                        pltpu.VMEM((1, BATCH), jnp.float32)],
    )(state, C2)


def kernel(story, query, A0, C0, C1, C2, TA, TC):
    idx_story = story.astype(jnp.int32).reshape(NSEG * SEN)
    idx_query = query.astype(jnp.int32).reshape(BATCH * SEN)
    ws = _pos_weights(SEN, EMBED)
    ws2 = jnp.concatenate([ws, ws], axis=1)
    AB0 = jnp.concatenate([A0, C0], axis=1)
    AB1 = jnp.concatenate([C1, C2], axis=1)
    p0, p1, pq = _sc_pool()(idx_story, idx_query, ws2, AB0, AB1)
    state = _hops(p0, p1, pq, TA, TC)
    return _logits(state, C2).T


# speedup vs baseline: 15.4196x; 1.1491x over previous
import functools

import jax
import jax.numpy as jnp
from jax import lax
from jax.experimental import pallas as pl
from jax.experimental.pallas import tpu as pltpu
from jax.experimental.pallas import tpu_sc as plsc

VOCAB = 100000
EMBED = 64
MEM = 50
SEN = 20
BATCH = 1024
NSEG = BATCH * MEM

NC, NS = 2, 16
NW = NC * NS
SEG_PER_W = NSEG // NW
BANK_SEGS = 16
BANK_ROWS = BANK_SEGS * SEN
GROWS = 80
GPB = BANK_ROWS // GROWS
NBATCH = SEG_PER_W // BANK_SEGS
Q_PER_W = BATCH // NW


def _pos_weights(J, d):
    j = jnp.arange(J, dtype=jnp.float32)[:, None]
    k = jnp.arange(d, dtype=jnp.float32)[None, :]
    return 1.0 - (j + 1.0) / J - (k + 1.0) / d * (1.0 - 2.0 * (j + 1.0) / J)


def _sc_pool_body(idx_story, idx_query, ws2, AB0, AB1,
                  out0, out1, outQ,
                  idx_v, rows0, rows1, pool_v, ws_v, sem0, sem1):
    wid = lax.axis_index("s") * NC + lax.axis_index("c")
    pltpu.sync_copy(ws2, ws_v)
    pltpu.sync_copy(idx_story.at[pl.ds(wid * (SEG_PER_W * SEN),
                                       SEG_PER_W * SEN)], idx_v)
    rows = (rows0, rows1)
    sems = (sem0, sem1)

    def fire(tbl, bank, t):
        for i in range(GPB):
            pltpu.async_copy(
                tbl.at[idx_v.at[pl.ds(t * BANK_ROWS + i * GROWS, GROWS)]],
                rows[bank].at[pl.ds(i * GROWS, GROWS)], sems[bank])

    def drain(tbl, bank):
        pltpu.make_async_copy(tbl.at[pl.ds(0, BANK_ROWS)], rows[bank],
                              sems[bank]).wait()

    def compute_write(bank, out_hbm, seg_off, n_seg):
        rv = rows[bank]
        for k8 in range(8):
            sl = pl.ds(k8 * 16, 16)
            wsr = [ws_v[s, sl] for s in range(SEN)]

            def seg_body(c, _):
                base = c * SEN
                terms = [rv[base + s, sl] * wsr[s] for s in range(SEN)]
                while len(terms) > 1:
                    nxt = [terms[i] + terms[i + 1]
                           for i in range(0, len(terms) - 1, 2)]
                    if len(terms) % 2:
                        nxt.append(terms[-1])
                    terms = nxt
                pool_v[c, sl] = terms[0]
                return 0

            lax.fori_loop(0, n_seg, seg_body, 0)
        pltpu.sync_copy(pool_v.at[pl.ds(0, n_seg)],
                        out_hbm.at[pl.ds(seg_off, n_seg)])

    for tbl, out_hbm in ((AB0, out0), (AB1, out1)):
        fire(tbl, 0, 0)

        def body2(j2, _, tbl=tbl, out_hbm=out_hbm):
            for b2 in range(2):
                t = j2 * 2 + b2

                @pl.when(t + 1 < NBATCH)
                def _(tbl=tbl, b2=b2, t=t):
                    fire(tbl, (b2 + 1) % 2, t + 1)

                drain(tbl, b2)
                compute_write(b2, out_hbm,
                              wid * SEG_PER_W + t * BANK_SEGS, BANK_SEGS)
            return 0

        lax.fori_loop(0, NBATCH // 2, body2, 0)

    pltpu.sync_copy(idx_query.at[pl.ds(wid * (Q_PER_W * SEN),
                                       Q_PER_W * SEN)],
                    idx_v.at[pl.ds(0, Q_PER_W * SEN)])
    for b2 in range(2):
        fire(AB0, b2, b2)
    for b2 in range(2):
        drain(AB0, b2)
        compute_write(b2, outQ, wid * Q_PER_W + b2 * BANK_SEGS, BANK_SEGS)


@functools.cache
def _sc_pool():
    return pl.kernel(
        _sc_pool_body,
        mesh=plsc.VectorSubcoreMesh(core_axis_name="c", subcore_axis_name="s"),
        out_type=[jax.ShapeDtypeStruct((NSEG, 128), jnp.float32)] * 2
        + [jax.ShapeDtypeStruct((BATCH, 128), jnp.float32)],
        scratch_types=[
            pltpu.VMEM((SEG_PER_W * SEN,), jnp.int32),
            pltpu.VMEM((BANK_ROWS, 128), jnp.float32),
            pltpu.VMEM((BANK_ROWS, 128), jnp.float32),
            pltpu.VMEM((BANK_SEGS, 128), jnp.float32),
            pltpu.VMEM((SEN, 128), jnp.float32),
            pltpu.SemaphoreType.DMA,
            pltpu.SemaphoreType.DMA,
        ],
    )


RB = 128


def _hops_body(p0_ref, p1_ref, pq_ref, ta_ref, tc_ref, out_ref):
    P0 = p0_ref[...]
    P1 = p1_ref[...]
    state = pq_ref[...][:, :EMBED]
    ta = ta_ref[...]
    tc = tc_ref[...]
    halves = (P0[..., :EMBED], P0[..., EMBED:], P1[..., :EMBED],
              P1[..., EMBED:])
    for i in range(3):
        mem = halves[i] + ta
        outp = halves[i + 1] + tc
        logits = jnp.sum(mem * state[:, None, :], axis=-1)
        m = jnp.max(logits, axis=-1, keepdims=True)
        e = jnp.exp(logits - m)
        p = e / jnp.sum(e, axis=-1, keepdims=True)
        state = state + jnp.sum(p[:, :, None] * outp, axis=1)
    out_ref[...] = state


def _hops(p0, p1, pq, TA, TC):
    pool_spec = pl.BlockSpec((RB, MEM, 128), lambda i: (i, 0, 0))
    p0 = p0.reshape(BATCH, MEM, 128)
    p1 = p1.reshape(BATCH, MEM, 128)
    return pl.pallas_call(
        _hops_body,
        grid=(BATCH // RB,),
        in_specs=[pool_spec, pool_spec,
                  pl.BlockSpec((RB, 128), lambda i: (i, 0)),
                  pl.BlockSpec((MEM, EMBED), lambda i: (0, 0)),
                  pl.BlockSpec((MEM, EMBED), lambda i: (0, 0))],
        out_specs=pl.BlockSpec((RB, EMBED), lambda i: (i, 0)),
        out_shape=jax.ShapeDtypeStruct((BATCH, EMBED), jnp.float32),
    )(p0, p1, pq, TA, TC)


VB = 2048
NVB = -(-VOCAB // VB)


def _logits_body(state_ref, c2_ref, out_ref, m_scr, s_scr):
    ph = pl.program_id(0)
    j = pl.program_id(1)
    x = lax.dot_general(c2_ref[...], state_ref[...],
                        (((1,), (1,)), ((), ())),
                        preferred_element_type=jnp.float32)

    @pl.when(ph == 0)
    def _():
        row = j * VB + lax.broadcasted_iota(jnp.int32, x.shape, 0)
        xm = jnp.where(row < VOCAB, x, -jnp.inf)

        @pl.when(j == 0)
        def _():
            m_scr[...] = jnp.full_like(m_scr, -jnp.inf)
            s_scr[...] = jnp.zeros_like(s_scr)

        m_old = m_scr[...]
        m_new = jnp.maximum(m_old, jnp.max(xm, axis=0, keepdims=True))
        s_scr[...] = s_scr[...] * jnp.exp(m_old - m_new) + \
            jnp.sum(jnp.exp(xm - m_new), axis=0, keepdims=True)
        m_scr[...] = m_new

    @pl.when(ph == 1)
    def _():
        out_ref[...] = x - (m_scr[...] + jnp.log(s_scr[...]))


def _logits(state, C2):
    return pl.pallas_call(
        _logits_body,
        grid=(2, NVB),
        in_specs=[pl.BlockSpec((BATCH, EMBED), lambda ph, j: (0, 0)),
                  pl.BlockSpec((VB, EMBED), lambda ph, j: (j, 0))],
        out_specs=pl.BlockSpec((VB, BATCH), lambda ph, j: (ph * j, 0)),
        out_shape=jax.ShapeDtypeStruct((VOCAB, BATCH), jnp.float32),
        scratch_shapes=[pltpu.VMEM((1, BATCH), jnp.float32),
                        pltpu.VMEM((1, BATCH), jnp.float32)],
    )(state, C2)


def kernel(story, query, A0, C0, C1, C2, TA, TC):
    idx_story = story.astype(jnp.int32).reshape(NSEG * SEN)
    idx_query = query.astype(jnp.int32).reshape(BATCH * SEN)
    ws = _pos_weights(SEN, EMBED)
    ws2 = jnp.concatenate([ws, ws], axis=1)
    AB0 = jnp.concatenate([A0, C0], axis=1)
    AB1 = jnp.concatenate([C1, C2], axis=1)
    p0, p1, pq = _sc_pool()(idx_story, idx_query, ws2, AB0, AB1)
    state = _hops(p0, p1, pq, TA, TC)
    return _logits(state, C2).T


# speedup vs baseline: 19.5311x; 1.2666x over previous
import functools

import jax
import jax.numpy as jnp
from jax import lax
from jax.experimental import pallas as pl
from jax.experimental.pallas import tpu as pltpu
from jax.experimental.pallas import tpu_sc as plsc

VOCAB = 100000
EMBED = 64
MEM = 50
MEMP = 56
SEN = 20
BATCH = 1024
NSEG = BATCH * MEM

NC, NS = 2, 16
NW = NC * NS
SEG_PER_W = NSEG // NW
B_PER_W = BATCH // NW
BANK_SEGS = 10
BANK_ROWS = BANK_SEGS * SEN
NBATCH = SEG_PER_W // BANK_SEGS
QBANK_SEGS = 8
Q_PER_W = BATCH // NW


def _pos_weights(J, d):
    j = jnp.arange(J, dtype=jnp.float32)[:, None]
    k = jnp.arange(d, dtype=jnp.float32)[None, :]
    return 1.0 - (j + 1.0) / J - (k + 1.0) / d * (1.0 - 2.0 * (j + 1.0) / J)


def _sc_pool_body(idx_story, idx_query, ws2, ABCD,
                  out0, out1, outQ,
                  idx_v, rows0, rows1, pool0, pool1, ws_v, sem0, sem1):
    wid = lax.axis_index("s") * NC + lax.axis_index("c")
    pltpu.sync_copy(ws2, ws_v)
    pltpu.sync_copy(idx_story.at[pl.ds(wid * (SEG_PER_W * SEN),
                                       SEG_PER_W * SEN)], idx_v)
    rows = (rows0, rows1)
    sems = (sem0, sem1)

    def fire(bank, t):
        for off, n in ((0, 104), (104, 96)):
            pltpu.async_copy(
                ABCD.at[idx_v.at[pl.ds(t * BANK_ROWS + off, n)]],
                rows[bank].at[pl.ds(off, n)], sems[bank])

    def drain(bank, nrows=BANK_ROWS):
        pltpu.make_async_copy(ABCD.at[pl.ds(0, nrows)],
                              rows[bank].at[pl.ds(0, nrows)],
                              sems[bank]).wait()

    def pool_bank(bank, m0, n_seg, both):
        rv = rows[bank]
        for k8 in range(8):
            sl = pl.ds(k8 * 16, 16)
            wsr = [ws_v[s, sl] for s in range(SEN)]

            def tree(terms):
                while len(terms) > 1:
                    nxt = [terms[i] + terms[i + 1]
                           for i in range(0, len(terms) - 1, 2)]
                    if len(terms) % 2:
                        nxt.append(terms[-1])
                    terms = nxt
                return terms[0]

            def seg_body(c, _):
                base = c * SEN
                los, his = [], []
                for s in range(SEN):
                    w = rv[base + s, sl]
                    flo = plsc.bitcast(lax.shift_left(w, 16), jnp.float32)
                    los.append(flo * wsr[s])
                    if both:
                        fhi = plsc.bitcast(w, jnp.float32)
                        his.append(fhi * wsr[s])
                pool0[m0 + c, sl] = tree(los)
                if both:
                    pool1[m0 + c, sl] = tree(his)
                return 0

            lax.fori_loop(0, n_seg, seg_body, 0)

    fire(0, 0)

    def body2(j2, _):
        for b2 in range(2):
            t = j2 * 2 + b2

            @pl.when(t + 1 < NBATCH)
            def _(b2=b2, t=t):
                fire((b2 + 1) % 2, t + 1)

            drain(b2)
            pool_bank(b2, lax.rem(t, 5) * BANK_SEGS, BANK_SEGS, True)

            @pl.when(lax.rem(t, 5) == 4)
            def _(t=t):
                rowoff = (wid * B_PER_W + lax.div(t, 5)) * MEMP
                pltpu.sync_copy(pool0, out0.at[pl.ds(rowoff, MEMP)])
                pltpu.sync_copy(pool1, out1.at[pl.ds(rowoff, MEMP)])
        return 0

    lax.fori_loop(0, NBATCH // 2, body2, 0)

    pltpu.sync_copy(idx_query.at[pl.ds(wid * (Q_PER_W * SEN),
                                       Q_PER_W * SEN)],
                    idx_v.at[pl.ds(0, Q_PER_W * SEN)])
    qrows = QBANK_SEGS * SEN

    def qfire(bank, qb):
        for i in range(2):
            pltpu.async_copy(
                ABCD.at[idx_v.at[pl.ds(qb * qrows + i * 80, 80)]],
                rows[bank].at[pl.ds(i * 80, 80)], sems[bank])

    qfire(0, 0)
    qfire(1, 1)
    for qb in range(4):
        bank = qb % 2
        drain(bank, qrows)
        pool_bank(bank, 0, QBANK_SEGS, False)
        pltpu.sync_copy(pool0.at[pl.ds(0, QBANK_SEGS)],
                        outQ.at[pl.ds(wid * Q_PER_W + qb * QBANK_SEGS,
                                      QBANK_SEGS)])
        if qb + 2 < 4:
            qfire(bank, qb + 2)


@functools.cache
def _sc_pool():
    return pl.kernel(
        _sc_pool_body,
        mesh=plsc.VectorSubcoreMesh(core_axis_name="c", subcore_axis_name="s"),
        out_type=[jax.ShapeDtypeStruct((BATCH * MEMP, 128), jnp.float32)] * 2
        + [jax.ShapeDtypeStruct((BATCH, 128), jnp.float32)],
        scratch_types=[
            pltpu.VMEM((SEG_PER_W * SEN,), jnp.int32),
            pltpu.VMEM((BANK_ROWS, 128), jnp.int32),
            pltpu.VMEM((BANK_ROWS, 128), jnp.int32),
            pltpu.VMEM((MEMP, 128), jnp.float32),
            pltpu.VMEM((MEMP, 128), jnp.float32),
            pltpu.VMEM((SEN, 128), jnp.float32),
            pltpu.SemaphoreType.DMA,
            pltpu.SemaphoreType.DMA,
        ],
        compiler_params=pltpu.CompilerParams(needs_layout_passes=False),
    )


RB = 128


def _hops_body(p0_ref, p1_ref, pq_ref, ta_ref, tc_ref, out_ref):
    P0 = p0_ref[...][:, :MEM, :]
    P1 = p1_ref[...][:, :MEM, :]
    state = pq_ref[...][:, :EMBED]
    ta = ta_ref[...]
    tc = tc_ref[...]
    halves = (P0[..., :EMBED], P0[..., EMBED:], P1[..., :EMBED],
              P1[..., EMBED:])
    for i in range(3):
        mem = halves[i] + ta
        outp = halves[i + 1] + tc
        logits = jnp.sum(mem * state[:, None, :], axis=-1)
        m = jnp.max(logits, axis=-1, keepdims=True)
        e = jnp.exp(logits - m)
        p = e / jnp.sum(e, axis=-1, keepdims=True)
        state = state + jnp.sum(p[:, :, None] * outp, axis=1)
    out_ref[...] = state


def _hops(p0, p1, pq, TA, TC):
    pool_spec = pl.BlockSpec((RB, MEMP, 128), lambda i: (i, 0, 0))
    return pl.pallas_call(
        _hops_body,
        grid=(BATCH // RB,),
        in_specs=[pool_spec, pool_spec,
                  pl.BlockSpec((RB, 128), lambda i: (i, 0)),
                  pl.BlockSpec((MEM, EMBED), lambda i: (0, 0)),
                  pl.BlockSpec((MEM, EMBED), lambda i: (0, 0))],
        out_specs=pl.BlockSpec((RB, EMBED), lambda i: (i, 0)),
        out_shape=jax.ShapeDtypeStruct((BATCH, EMBED), jnp.float32),
    )(p0, p1, pq, TA, TC)


VB = 2048
NVB = -(-VOCAB // VB)


def _logits_body(state_ref, c2_ref, out_ref, m_scr, s_scr):
    ph = pl.program_id(0)
    j = pl.program_id(1)

    @pl.when(ph == 0)
    def _():
        x = lax.dot_general(c2_ref[...].astype(jnp.bfloat16),
                            state_ref[...].astype(jnp.bfloat16),
                            (((1,), (1,)), ((), ())),
                            preferred_element_type=jnp.float32)
        row = j * VB + lax.broadcasted_iota(jnp.int32, x.shape, 0)
        xm = jnp.where(row < VOCAB, x, -jnp.inf)

        @pl.when(j == 0)
        def _():
            m_scr[...] = jnp.full_like(m_scr, -jnp.inf)
            s_scr[...] = jnp.zeros_like(s_scr)

        m_old = m_scr[...]
        m_new = jnp.maximum(m_old, jnp.max(xm, axis=0, keepdims=True))
        s_scr[...] = s_scr[...] * jnp.exp(m_old - m_new) + \
            jnp.sum(jnp.exp(xm - m_new), axis=0, keepdims=True)
        m_scr[...] = m_new

    @pl.when(ph == 1)
    def _():
        x = lax.dot_general(c2_ref[...], state_ref[...],
                            (((1,), (1,)), ((), ())),
                            preferred_element_type=jnp.float32)
        out_ref[...] = x - (m_scr[...] + jnp.log(s_scr[...]))


def _logits(state, C2):
    return pl.pallas_call(
        _logits_body,
        grid=(2, NVB),
        in_specs=[pl.BlockSpec((BATCH, EMBED), lambda ph, j: (0, 0)),
                  pl.BlockSpec((VB, EMBED), lambda ph, j: (j, 0))],
        out_specs=pl.BlockSpec((VB, BATCH), lambda ph, j: (ph * j, 0)),
        out_shape=jax.ShapeDtypeStruct((VOCAB, BATCH), jnp.float32),
        scratch_shapes=[pltpu.VMEM((1, BATCH), jnp.float32),
                        pltpu.VMEM((1, BATCH), jnp.float32)],
    )(state, C2)


def _pack_tables(A0, C0, C1, C2):
    lo = lax.bitcast_convert_type(
        jnp.concatenate([A0, C0], axis=1).astype(jnp.bfloat16), jnp.uint16)
    hi = lax.bitcast_convert_type(
        jnp.concatenate([C1, C2], axis=1).astype(jnp.bfloat16), jnp.uint16)
    word = lo.astype(jnp.uint32) | (hi.astype(jnp.uint32) << 16)
    return lax.bitcast_convert_type(word, jnp.int32)


def kernel(story, query, A0, C0, C1, C2, TA, TC):
    idx_story = story.astype(jnp.int32).reshape(NSEG * SEN)
    idx_query = query.astype(jnp.int32).reshape(BATCH * SEN)
    ws = _pos_weights(SEN, EMBED)
    ws2 = jnp.concatenate([ws, ws], axis=1)
    ABCD = _pack_tables(A0, C0, C1, C2)
    p0, p1, pq = _sc_pool()(idx_story, idx_query, ws2, ABCD)
    state = _hops(p0.reshape(BATCH, MEMP, 128),
                  p1.reshape(BATCH, MEMP, 128), pq, TA, TC)
    return _logits(state, C2).T
